# Initial kernel scaffold; baseline (speedup 1.0000x reference)
import jax
import jax.numpy as jnp
from jax.experimental import pallas as pl


def kernel(user_x, repost_edge_index, follow_edge_index, user_batch):
    raise NotImplementedError("write your pallas kernel here")


# speedup vs baseline: 12.1011x; 12.1011x over previous
import functools

import jax
import jax.numpy as jnp
from jax import lax
from jax.experimental import pallas as pl
from jax.experimental.pallas import tpu as pltpu
from jax.experimental.pallas import tpu_sc as plsc

N_NODES = 10000
N_EDGES = 160000
D_FEAT = 128
TOTAL = N_NODES + 3 * N_EDGES

CHUNK = 112
NCHUNKS = TOTAL // CHUNK
NW = 32
_Q, _R = divmod(NCHUNKS, NW)

_mesh = plsc.VectorSubcoreMesh(core_axis_name="c", subcore_axis_name="s")


@functools.partial(
    pl.kernel,
    mesh=_mesh,
    out_type=[
        jax.ShapeDtypeStruct((TOTAL, D_FEAT), jnp.float32),
        jax.ShapeDtypeStruct((TOTAL,), jnp.int32),
    ],
    scratch_types=[
        pltpu.VMEM((CHUNK,), jnp.int32),
        pltpu.VMEM((CHUNK, D_FEAT), jnp.float32),
        pltpu.VMEM((CHUNK,), jnp.int32),
        pltpu.SemaphoreType.DMA,
    ],
)
def _gather_sc(x_hbm, idx_hbm, b_hbm, out_hbm, bout_hbm,
               idx_v, rows_v, vals_v, sem):
    q = jnp.int32(_Q)
    r = jnp.int32(_R)
    ch = jnp.int32(CHUNK)
    w = (lax.axis_index("s") * jnp.int32(2) + lax.axis_index("c")).astype(jnp.int32)
    cnt = q + jnp.where(w < r, jnp.int32(1), jnp.int32(0))
    start = w * q + jnp.minimum(w, r)

    def body(k, carry):
        base = (start + k) * ch
        pltpu.sync_copy(idx_hbm.at[pl.ds(base, CHUNK)], idx_v)
        pltpu.async_copy(x_hbm.at[idx_v], rows_v, sem).wait()
        pltpu.async_copy(b_hbm.at[idx_v], vals_v, sem).wait()
        pltpu.sync_copy(rows_v, out_hbm.at[pl.ds(base, CHUNK)])
        pltpu.sync_copy(vals_v, bout_hbm.at[pl.ds(base, CHUNK)])
        return carry

    lax.fori_loop(jnp.int32(0), cnt, body, jnp.int32(0))


def kernel(user_x, repost_edge_index, follow_edge_index, user_batch):
    idx_all = jnp.concatenate([
        jnp.arange(N_NODES, dtype=jnp.int32),
        repost_edge_index[0].astype(jnp.int32),
        repost_edge_index[1].astype(jnp.int32),
        follow_edge_index[1].astype(jnp.int32),
    ])
    batch_i32 = user_batch.astype(jnp.int32)
    out, bvals = _gather_sc(user_x, idx_all, batch_i32)
    e0 = N_NODES
    sender_batch = bvals[e0:e0 + N_EDGES].astype(user_batch.dtype)
    receiver_batch = bvals[e0 + N_EDGES:e0 + 2 * N_EDGES].astype(user_batch.dtype)
    follower_batch = bvals[e0 + 2 * N_EDGES:].astype(user_batch.dtype)
    return out, sender_batch, receiver_batch, follower_batch


# speedup vs baseline: 21.3820x; 1.7670x over previous
import functools

import jax
import jax.numpy as jnp
from jax import lax
from jax.experimental import pallas as pl
from jax.experimental.pallas import tpu as pltpu
from jax.experimental.pallas import tpu_sc as plsc

N_NODES = 10000
N_EDGES = 160000
D_FEAT = 128
TOTAL = N_NODES + 3 * N_EDGES

CHUNK = 128
RCHUNKS = -(-TOTAL // CHUNK)
LAST_BASE = TOTAL - CHUNK
NW = 32
CPW = 120
KG = 2
NSLOT = 3
NITER = CPW // (KG * NSLOT)
PRELOAD = CPW * CHUNK
W31_MAIN = RCHUNKS * CHUNK - (NW - 1) * PRELOAD
GROUP = KG * CHUNK

_mesh = plsc.VectorSubcoreMesh(core_axis_name="c", subcore_axis_name="s")


@functools.partial(
    pl.kernel,
    mesh=_mesh,
    out_type=[
        jax.ShapeDtypeStruct((TOTAL, D_FEAT), jnp.float32),
        jax.ShapeDtypeStruct((TOTAL,), jnp.int32),
    ],
    scratch_types=[
        pltpu.VMEM((PRELOAD,), jnp.int32),
        pltpu.VMEM((NSLOT, GROUP, D_FEAT), jnp.float32),
        pltpu.VMEM((NSLOT * GROUP,), jnp.int32),
        pltpu.SemaphoreType.DMA,
        pltpu.SemaphoreType.DMA,
        pltpu.SemaphoreType.DMA,
        pltpu.SemaphoreType.DMA,
        pltpu.SemaphoreType.DMA,
        pltpu.SemaphoreType.DMA,
    ],
)
def _gather_sc(x_hbm, idx_hbm, b_hbm, out_hbm, bout_hbm,
               idx_v, rows_v, vals_v,
               gsem0, gsem1, gsem2, wsem0, wsem1, wsem2):
    gsems = (gsem0, gsem1, gsem2)
    wsems = (wsem0, wsem1, wsem2)
    w = (lax.axis_index("s") * jnp.int32(2) + lax.axis_index("c")).astype(jnp.int32)
    w0 = w * jnp.int32(CPW)

    @pl.when(w < jnp.int32(NW - 1))
    def _():
        pltpu.sync_copy(idx_hbm.at[pl.ds(w * jnp.int32(PRELOAD), PRELOAD)], idx_v)

    @pl.when(w == jnp.int32(NW - 1))
    def _():
        pltpu.sync_copy(idx_hbm.at[pl.ds(jnp.int32((NW - 1) * PRELOAD), W31_MAIN)],
                        idx_v.at[pl.ds(jnp.int32(0), W31_MAIN)])
        pltpu.sync_copy(idx_hbm.at[pl.ds(jnp.int32(0), PRELOAD - W31_MAIN)],
                        idx_v.at[pl.ds(jnp.int32(W31_MAIN), PRELOAD - W31_MAIN)])

    def chunk_addr(c):
        creal = jnp.where(c < jnp.int32(RCHUNKS), c, c - jnp.int32(RCHUNKS))
        base = jnp.minimum(creal * jnp.int32(CHUNK), jnp.int32(LAST_BASE))
        off = jnp.where(c < jnp.int32(RCHUNKS),
                        base - w * jnp.int32(PRELOAD),
                        base + jnp.int32(W31_MAIN))
        return base, off

    def fire_gathers(s, g):
        for j in range(KG):
            c = w0 + g * jnp.int32(KG) + jnp.int32(j)
            _, off = chunk_addr(c)
            iv = idx_v.at[pl.ds(off, CHUNK)]
            pltpu.async_copy(x_hbm.at[iv], rows_v.at[jnp.int32(s), pl.ds(jnp.int32(j * CHUNK), CHUNK)],
                             gsems[s])
            pltpu.async_copy(b_hbm.at[iv], vals_v.at[pl.ds(jnp.int32(s * GROUP + j * CHUNK), CHUNK)],
                             gsems[s])

    def drain_gathers_fire_writes(s, g):
        for j in range(KG):
            c = w0 + g * jnp.int32(KG) + jnp.int32(j)
            _, off = chunk_addr(c)
            iv = idx_v.at[pl.ds(off, CHUNK)]
            pltpu.make_async_copy(x_hbm.at[iv],
                                  rows_v.at[jnp.int32(s), pl.ds(jnp.int32(j * CHUNK), CHUNK)],
                                  gsems[s]).wait()
            pltpu.make_async_copy(b_hbm.at[iv],
                                  vals_v.at[pl.ds(jnp.int32(s * GROUP + j * CHUNK), CHUNK)],
                                  gsems[s]).wait()
        for j in range(KG):
            c = w0 + g * jnp.int32(KG) + jnp.int32(j)
            base, _ = chunk_addr(c)
            pltpu.async_copy(rows_v.at[jnp.int32(s), pl.ds(jnp.int32(j * CHUNK), CHUNK)],
                             out_hbm.at[pl.ds(base, CHUNK)], wsems[s])
            pltpu.async_copy(vals_v.at[pl.ds(jnp.int32(s * GROUP + j * CHUNK), CHUNK)],
                             bout_hbm.at[pl.ds(base, CHUNK)], wsems[s])

    def drain_writes(s):
        for j in range(KG):
            pltpu.make_async_copy(rows_v.at[jnp.int32(s), pl.ds(jnp.int32(j * CHUNK), CHUNK)],
                                  out_hbm.at[pl.ds(jnp.int32(0), CHUNK)],
                                  wsems[s]).wait()
            pltpu.make_async_copy(vals_v.at[pl.ds(jnp.int32(s * GROUP + j * CHUNK), CHUNK)],
                                  bout_hbm.at[pl.ds(jnp.int32(0), CHUNK)],
                                  wsems[s]).wait()

    def body(t, carry):
        g0 = t * jnp.int32(NSLOT)
        for s in range(NSLOT):
            @pl.when(t > jnp.int32(0))
            def _(s=s):
                drain_writes(s)
            fire_gathers(s, g0 + jnp.int32(s))
        for s in range(NSLOT):
            drain_gathers_fire_writes(s, g0 + jnp.int32(s))
        return carry

    lax.fori_loop(jnp.int32(0), jnp.int32(NITER), body, jnp.int32(0))
    for s in range(NSLOT):
        drain_writes(s)


def kernel(user_x, repost_edge_index, follow_edge_index, user_batch):
    idx_all = jnp.concatenate([
        jnp.arange(N_NODES, dtype=jnp.int32),
        repost_edge_index[0].astype(jnp.int32),
        repost_edge_index[1].astype(jnp.int32),
        follow_edge_index[1].astype(jnp.int32),
    ])
    batch_i32 = user_batch.astype(jnp.int32)
    out, bvals = _gather_sc(user_x, idx_all, batch_i32)
    e0 = N_NODES
    sender_batch = bvals[e0:e0 + N_EDGES].astype(user_batch.dtype)
    receiver_batch = bvals[e0 + N_EDGES:e0 + 2 * N_EDGES].astype(user_batch.dtype)
    follower_batch = bvals[e0 + 2 * N_EDGES:].astype(user_batch.dtype)
    return out, sender_batch, receiver_batch, follower_batch


# speedup vs baseline: 26.7838x; 1.2526x over previous
import functools

import jax
import jax.numpy as jnp
from jax import lax
from jax.experimental import pallas as pl
from jax.experimental.pallas import tpu as pltpu
from jax.experimental.pallas import tpu_sc as plsc

N_NODES = 10000
N_EDGES = 160000
D_FEAT = 128
TOTAL = N_NODES + 3 * N_EDGES

CHUNK = 128
RCHUNKS = -(-TOTAL // CHUNK)
LAST_BASE = TOTAL - CHUNK
NW = 32
CPW = 120
KG = 2
NSLOT = 3
NITER = CPW // (KG * NSLOT)
PRELOAD = CPW * CHUNK
W31_MAIN = RCHUNKS * CHUNK - (NW - 1) * PRELOAD
GROUP = KG * CHUNK

_mesh = plsc.VectorSubcoreMesh(core_axis_name="c", subcore_axis_name="s")


@functools.partial(
    pl.kernel,
    mesh=_mesh,
    compiler_params=pltpu.CompilerParams(needs_layout_passes=False),
    out_type=[
        jax.ShapeDtypeStruct((TOTAL, D_FEAT), jnp.float32),
        jax.ShapeDtypeStruct((TOTAL,), jnp.int32),
    ],
    scratch_types=[
        pltpu.VMEM((PRELOAD,), jnp.int32),
        pltpu.VMEM((NSLOT, GROUP, D_FEAT), jnp.float32),
        pltpu.VMEM((NSLOT * GROUP,), jnp.int32),
        pltpu.VMEM((N_NODES,), jnp.int32),
        pltpu.SemaphoreType.DMA,
        pltpu.SemaphoreType.DMA,
        pltpu.SemaphoreType.DMA,
        pltpu.SemaphoreType.DMA,
        pltpu.SemaphoreType.DMA,
        pltpu.SemaphoreType.DMA,
    ],
)
def _gather_sc(x_hbm, idx_hbm, b_hbm, out_hbm, bout_hbm,
               idx_v, rows_v, vals_v, bt_v,
               gsem0, gsem1, gsem2, wsem0, wsem1, wsem2):
    gsems = (gsem0, gsem1, gsem2)
    wsems = (wsem0, wsem1, wsem2)
    w = (lax.axis_index("s") * jnp.int32(2) + lax.axis_index("c")).astype(jnp.int32)
    w0 = w * jnp.int32(CPW)

    pltpu.sync_copy(b_hbm, bt_v)

    @pl.when(w < jnp.int32(NW - 1))
    def _():
        pltpu.sync_copy(idx_hbm.at[pl.ds(w * jnp.int32(PRELOAD), PRELOAD)], idx_v)

    @pl.when(w == jnp.int32(NW - 1))
    def _():
        pltpu.sync_copy(idx_hbm.at[pl.ds(jnp.int32((NW - 1) * PRELOAD), W31_MAIN)],
                        idx_v.at[pl.ds(jnp.int32(0), W31_MAIN)])
        pltpu.sync_copy(idx_hbm.at[pl.ds(jnp.int32(0), PRELOAD - W31_MAIN)],
                        idx_v.at[pl.ds(jnp.int32(W31_MAIN), PRELOAD - W31_MAIN)])

    def chunk_addr(c):
        creal = jnp.where(c < jnp.int32(RCHUNKS), c, c - jnp.int32(RCHUNKS))
        base = jnp.minimum(creal * jnp.int32(CHUNK), jnp.int32(LAST_BASE))
        off = jnp.where(c < jnp.int32(RCHUNKS),
                        base - w * jnp.int32(PRELOAD),
                        base + jnp.int32(W31_MAIN))
        return base, off

    def fire_gathers(s, g):
        for j in range(KG):
            c = w0 + g * jnp.int32(KG) + jnp.int32(j)
            base, off = chunk_addr(c)
            iv = idx_v.at[pl.ds(off, CHUNK)]
            pltpu.async_copy(x_hbm.at[iv], rows_v.at[jnp.int32(s), pl.ds(jnp.int32(j * CHUNK), CHUNK)],
                             gsems[s])
            for v in range(CHUNK // 16):
                i16 = idx_v[pl.ds(off + jnp.int32(v * 16), 16)]
                vals_v[pl.ds(jnp.int32(s * GROUP + j * CHUNK + v * 16), 16)] = \
                    plsc.load_gather(bt_v, [i16])
            pltpu.async_copy(vals_v.at[pl.ds(jnp.int32(s * GROUP + j * CHUNK), CHUNK)],
                             bout_hbm.at[pl.ds(base, CHUNK)], wsems[s])

    def drain_gathers_fire_writes(s, g):
        for j in range(KG):
            c = w0 + g * jnp.int32(KG) + jnp.int32(j)
            _, off = chunk_addr(c)
            iv = idx_v.at[pl.ds(off, CHUNK)]
            pltpu.make_async_copy(x_hbm.at[iv],
                                  rows_v.at[jnp.int32(s), pl.ds(jnp.int32(j * CHUNK), CHUNK)],
                                  gsems[s]).wait()
        for j in range(KG):
            c = w0 + g * jnp.int32(KG) + jnp.int32(j)
            base, _ = chunk_addr(c)
            pltpu.async_copy(rows_v.at[jnp.int32(s), pl.ds(jnp.int32(j * CHUNK), CHUNK)],
                             out_hbm.at[pl.ds(base, CHUNK)], wsems[s])

    def drain_writes(s):
        for j in range(KG):
            pltpu.make_async_copy(rows_v.at[jnp.int32(s), pl.ds(jnp.int32(j * CHUNK), CHUNK)],
                                  out_hbm.at[pl.ds(jnp.int32(0), CHUNK)],
                                  wsems[s]).wait()
            pltpu.make_async_copy(vals_v.at[pl.ds(jnp.int32(s * GROUP + j * CHUNK), CHUNK)],
                                  bout_hbm.at[pl.ds(jnp.int32(0), CHUNK)],
                                  wsems[s]).wait()

    def body(t, carry):
        g0 = t * jnp.int32(NSLOT)
        for s in range(NSLOT):
            @pl.when(t > jnp.int32(0))
            def _(s=s):
                drain_writes(s)
            fire_gathers(s, g0 + jnp.int32(s))
        for s in range(NSLOT):
            drain_gathers_fire_writes(s, g0 + jnp.int32(s))
        return carry

    lax.fori_loop(jnp.int32(0), jnp.int32(NITER), body, jnp.int32(0))
    for s in range(NSLOT):
        drain_writes(s)


def kernel(user_x, repost_edge_index, follow_edge_index, user_batch):
    idx_all = jnp.concatenate([
        jnp.arange(N_NODES, dtype=jnp.int32),
        repost_edge_index[0].astype(jnp.int32),
        repost_edge_index[1].astype(jnp.int32),
        follow_edge_index[1].astype(jnp.int32),
    ])
    batch_i32 = user_batch.astype(jnp.int32)
    out, bvals = _gather_sc(user_x, idx_all, batch_i32)
    e0 = N_NODES
    sender_batch = bvals[e0:e0 + N_EDGES].astype(user_batch.dtype)
    receiver_batch = bvals[e0 + N_EDGES:e0 + 2 * N_EDGES].astype(user_batch.dtype)
    follower_batch = bvals[e0 + 2 * N_EDGES:].astype(user_batch.dtype)
    return out, sender_batch, receiver_batch, follower_batch


# speedup vs baseline: 26.9958x; 1.0079x over previous
import functools

import jax
import jax.numpy as jnp
from jax import lax
from jax.experimental import pallas as pl
from jax.experimental.pallas import tpu as pltpu
from jax.experimental.pallas import tpu_sc as plsc

N_NODES = 10000
N_EDGES = 160000
D_FEAT = 128
TOTAL = N_NODES + 3 * N_EDGES

CHUNK = 128
RCHUNKS = -(-TOTAL // CHUNK)
LAST_BASE = TOTAL - CHUNK
NW = 32
CPW = 120
KG = 1
NSLOT = 6
NITER = CPW // (KG * NSLOT)
PRELOAD = CPW * CHUNK
W31_MAIN = RCHUNKS * CHUNK - (NW - 1) * PRELOAD
GROUP = KG * CHUNK

_mesh = plsc.VectorSubcoreMesh(core_axis_name="c", subcore_axis_name="s")


@functools.partial(
    pl.kernel,
    mesh=_mesh,
    compiler_params=pltpu.CompilerParams(needs_layout_passes=False),
    out_type=[
        jax.ShapeDtypeStruct((TOTAL, D_FEAT), jnp.float32),
        jax.ShapeDtypeStruct((TOTAL,), jnp.int32),
    ],
    scratch_types=[
        pltpu.VMEM((PRELOAD,), jnp.int32),
        pltpu.VMEM((NSLOT, GROUP, D_FEAT), jnp.float32),
        pltpu.VMEM((NSLOT * GROUP,), jnp.int32),
        pltpu.VMEM((N_NODES,), jnp.int32),
        pltpu.SemaphoreType.DMA,
        pltpu.SemaphoreType.DMA,
        pltpu.SemaphoreType.DMA,
        pltpu.SemaphoreType.DMA,
        pltpu.SemaphoreType.DMA,
        pltpu.SemaphoreType.DMA,
        pltpu.SemaphoreType.DMA,
        pltpu.SemaphoreType.DMA,
        pltpu.SemaphoreType.DMA,
        pltpu.SemaphoreType.DMA,
        pltpu.SemaphoreType.DMA,
        pltpu.SemaphoreType.DMA,
    ],
)
def _gather_sc(x_hbm, idx_hbm, b_hbm, out_hbm, bout_hbm,
               idx_v, rows_v, vals_v, bt_v,
               gsem0, gsem1, gsem2, gsem3, gsem4, gsem5,
               wsem0, wsem1, wsem2, wsem3, wsem4, wsem5):
    gsems = (gsem0, gsem1, gsem2, gsem3, gsem4, gsem5)
    wsems = (wsem0, wsem1, wsem2, wsem3, wsem4, wsem5)
    w = (lax.axis_index("s") * jnp.int32(2) + lax.axis_index("c")).astype(jnp.int32)
    w0 = w * jnp.int32(CPW)

    pltpu.sync_copy(b_hbm, bt_v)

    @pl.when(w < jnp.int32(NW - 1))
    def _():
        pltpu.sync_copy(idx_hbm.at[pl.ds(w * jnp.int32(PRELOAD), PRELOAD)], idx_v)

    @pl.when(w == jnp.int32(NW - 1))
    def _():
        pltpu.sync_copy(idx_hbm.at[pl.ds(jnp.int32((NW - 1) * PRELOAD), W31_MAIN)],
                        idx_v.at[pl.ds(jnp.int32(0), W31_MAIN)])
        pltpu.sync_copy(idx_hbm.at[pl.ds(jnp.int32(0), PRELOAD - W31_MAIN)],
                        idx_v.at[pl.ds(jnp.int32(W31_MAIN), PRELOAD - W31_MAIN)])

    def chunk_addr(c):
        creal = jnp.where(c < jnp.int32(RCHUNKS), c, c - jnp.int32(RCHUNKS))
        base = jnp.minimum(creal * jnp.int32(CHUNK), jnp.int32(LAST_BASE))
        off = jnp.where(c < jnp.int32(RCHUNKS),
                        base - w * jnp.int32(PRELOAD),
                        base + jnp.int32(W31_MAIN))
        return base, off

    def fire_gathers(s, g):
        for j in range(KG):
            c = w0 + g * jnp.int32(KG) + jnp.int32(j)
            base, off = chunk_addr(c)
            iv = idx_v.at[pl.ds(off, CHUNK)]
            pltpu.async_copy(x_hbm.at[iv], rows_v.at[jnp.int32(s), pl.ds(jnp.int32(j * CHUNK), CHUNK)],
                             gsems[s])
            for v in range(CHUNK // 16):
                i16 = idx_v[pl.ds(off + jnp.int32(v * 16), 16)]
                vals_v[pl.ds(jnp.int32(s * GROUP + j * CHUNK + v * 16), 16)] = \
                    plsc.load_gather(bt_v, [i16])
            pltpu.async_copy(vals_v.at[pl.ds(jnp.int32(s * GROUP + j * CHUNK), CHUNK)],
                             bout_hbm.at[pl.ds(base, CHUNK)], wsems[s])

    def drain_gathers_fire_writes(s, g):
        for j in range(KG):
            c = w0 + g * jnp.int32(KG) + jnp.int32(j)
            _, off = chunk_addr(c)
            iv = idx_v.at[pl.ds(off, CHUNK)]
            pltpu.make_async_copy(x_hbm.at[iv],
                                  rows_v.at[jnp.int32(s), pl.ds(jnp.int32(j * CHUNK), CHUNK)],
                                  gsems[s]).wait()
        for j in range(KG):
            c = w0 + g * jnp.int32(KG) + jnp.int32(j)
            base, _ = chunk_addr(c)
            pltpu.async_copy(rows_v.at[jnp.int32(s), pl.ds(jnp.int32(j * CHUNK), CHUNK)],
                             out_hbm.at[pl.ds(base, CHUNK)], wsems[s])

    def drain_writes(s):
        for j in range(KG):
            pltpu.make_async_copy(rows_v.at[jnp.int32(s), pl.ds(jnp.int32(j * CHUNK), CHUNK)],
                                  out_hbm.at[pl.ds(jnp.int32(0), CHUNK)],
                                  wsems[s]).wait()
            pltpu.make_async_copy(vals_v.at[pl.ds(jnp.int32(s * GROUP + j * CHUNK), CHUNK)],
                                  bout_hbm.at[pl.ds(jnp.int32(0), CHUNK)],
                                  wsems[s]).wait()

    def body(t, carry):
        g0 = t * jnp.int32(NSLOT)
        for s in range(NSLOT):
            @pl.when(t > jnp.int32(0))
            def _(s=s):
                drain_writes(s)
            fire_gathers(s, g0 + jnp.int32(s))
        for s in range(NSLOT):
            drain_gathers_fire_writes(s, g0 + jnp.int32(s))
        return carry

    lax.fori_loop(jnp.int32(0), jnp.int32(NITER), body, jnp.int32(0))
    for s in range(NSLOT):
        drain_writes(s)


def kernel(user_x, repost_edge_index, follow_edge_index, user_batch):
    idx_all = jnp.concatenate([
        jnp.arange(N_NODES, dtype=jnp.int32),
        repost_edge_index[0].astype(jnp.int32),
        repost_edge_index[1].astype(jnp.int32),
        follow_edge_index[1].astype(jnp.int32),
    ])
    batch_i32 = user_batch.astype(jnp.int32)
    out, bvals = _gather_sc(user_x, idx_all, batch_i32)
    e0 = N_NODES
    sender_batch = bvals[e0:e0 + N_EDGES].astype(user_batch.dtype)
    receiver_batch = bvals[e0 + N_EDGES:e0 + 2 * N_EDGES].astype(user_batch.dtype)
    follower_batch = bvals[e0 + 2 * N_EDGES:].astype(user_batch.dtype)
    return out, sender_batch, receiver_batch, follower_batch


# speedup vs baseline: 34.2266x; 1.2679x over previous
import functools

import jax
import jax.numpy as jnp
from jax import lax
from jax.experimental import pallas as pl
from jax.experimental.pallas import tpu as pltpu
from jax.experimental.pallas import tpu_sc as plsc

N_NODES = 10000
N_EDGES = 160000
D_FEAT = 128
TOTAL = N_NODES + 3 * N_EDGES

CHUNK = 128
RCHUNKS = -(-TOTAL // CHUNK)
LAST_BASE = TOTAL - CHUNK
NW = 32
CPW = 120
NSLOT = 3
NITER = CPW // NSLOT
TROWS = 632

_mesh = plsc.VectorSubcoreMesh(core_axis_name="c", subcore_axis_name="s")


@functools.partial(
    pl.kernel,
    mesh=_mesh,
    compiler_params=pltpu.CompilerParams(needs_layout_passes=False),
    out_type=[
        jax.ShapeDtypeStruct((TOTAL, D_FEAT), jnp.float32),
        jax.ShapeDtypeStruct((TOTAL,), jnp.int32),
    ],
    scratch_types=[
        pltpu.VMEM((NSLOT * CHUNK,), jnp.int32),
        pltpu.VMEM((NSLOT, CHUNK, D_FEAT), jnp.float32),
        pltpu.VMEM((NSLOT * CHUNK,), jnp.int32),
        pltpu.VMEM_SHARED((N_NODES, D_FEAT), jnp.float32),
        pltpu.VMEM_SHARED((N_NODES,), jnp.int32),
        pltpu.SemaphoreType.DMA,
        pltpu.SemaphoreType.DMA,
        pltpu.SemaphoreType.DMA,
        pltpu.SemaphoreType.DMA,
        pltpu.SemaphoreType.DMA,
        pltpu.SemaphoreType.DMA,
        pltpu.SemaphoreType.DMA,
        pltpu.SemaphoreType.DMA,
        pltpu.SemaphoreType.DMA,
    ],
)
def _gather_sc(x_hbm, idx_hbm, b_hbm, out_hbm, bout_hbm,
               idx_v, rows_v, vals_v, xs_sh, bt_sh,
               isem0, isem1, isem2, gsem0, gsem1, gsem2,
               wsem0, wsem1, wsem2):
    isems = (isem0, isem1, isem2)
    gsems = (gsem0, gsem1, gsem2)
    wsems = (wsem0, wsem1, wsem2)
    w = (lax.axis_index("s") * jnp.int32(2) + lax.axis_index("c")).astype(jnp.int32)
    w0 = w * jnp.int32(CPW)

    sid = lax.axis_index("s").astype(jnp.int32)
    rstart = sid * jnp.int32(TROWS)

    def bounce_bt(start, size):
        pltpu.sync_copy(b_hbm.at[pl.ds(start, size)],
                        idx_v.at[pl.ds(jnp.int32(0), size)])
        pltpu.sync_copy(idx_v.at[pl.ds(jnp.int32(0), size)],
                        bt_sh.at[pl.ds(start, size)])

    @pl.when(sid < jnp.int32(15))
    def _():
        pltpu.sync_copy(x_hbm.at[pl.ds(rstart, TROWS)],
                        xs_sh.at[pl.ds(rstart, TROWS)])
        bounce_bt(rstart, 384)
        bounce_bt(rstart + jnp.int32(384), TROWS - 384)

    @pl.when(sid == jnp.int32(15))
    def _():
        last = jnp.int32(15 * TROWS)
        pltpu.sync_copy(x_hbm.at[pl.ds(last, N_NODES - 15 * TROWS)],
                        xs_sh.at[pl.ds(last, N_NODES - 15 * TROWS)])
        bounce_bt(last, 384)
        bounce_bt(last + jnp.int32(384), N_NODES - 15 * TROWS - 384)

    plsc.subcore_barrier()

    def chunk_base(c):
        creal = jnp.where(c < jnp.int32(RCHUNKS), c, c - jnp.int32(RCHUNKS))
        return jnp.minimum(creal * jnp.int32(CHUNK), jnp.int32(LAST_BASE))

    def slot_refs(s):
        return (idx_v.at[pl.ds(jnp.int32(s * CHUNK), CHUNK)],
                rows_v.at[jnp.int32(s)],
                vals_v.at[pl.ds(jnp.int32(s * CHUNK), CHUNK)])

    def drain_writes(s):
        iv, rv, vv = slot_refs(s)
        pltpu.make_async_copy(rv, out_hbm.at[pl.ds(jnp.int32(0), CHUNK)],
                              wsems[s]).wait()
        pltpu.make_async_copy(vv, bout_hbm.at[pl.ds(jnp.int32(0), CHUNK)],
                              wsems[s]).wait()

    def body(t, carry):
        c0 = w0 + t * jnp.int32(NSLOT)
        for s in range(NSLOT):
            base = chunk_base(c0 + jnp.int32(s))
            iv, rv, vv = slot_refs(s)

            @pl.when(t > jnp.int32(0))
            def _(s=s):
                drain_writes(s)

            pltpu.async_copy(idx_hbm.at[pl.ds(base, CHUNK)], iv, isems[s])
        for s in range(NSLOT):
            iv, rv, vv = slot_refs(s)
            pltpu.make_async_copy(idx_hbm.at[pl.ds(jnp.int32(0), CHUNK)], iv,
                                  isems[s]).wait()
            pltpu.async_copy(xs_sh.at[iv], rv, gsems[s])
            pltpu.async_copy(bt_sh.at[iv], vv, gsems[s])
        for s in range(NSLOT):
            base = chunk_base(c0 + jnp.int32(s))
            iv, rv, vv = slot_refs(s)
            pltpu.make_async_copy(xs_sh.at[iv], rv, gsems[s]).wait()
            pltpu.make_async_copy(bt_sh.at[iv], vv, gsems[s]).wait()
            pltpu.async_copy(rv, out_hbm.at[pl.ds(base, CHUNK)], wsems[s])
            pltpu.async_copy(vv, bout_hbm.at[pl.ds(base, CHUNK)], wsems[s])
        return carry

    lax.fori_loop(jnp.int32(0), jnp.int32(NITER), body, jnp.int32(0))
    for s in range(NSLOT):
        drain_writes(s)


def kernel(user_x, repost_edge_index, follow_edge_index, user_batch):
    idx_all = jnp.concatenate([
        jnp.arange(N_NODES, dtype=jnp.int32),
        repost_edge_index[0].astype(jnp.int32),
        repost_edge_index[1].astype(jnp.int32),
        follow_edge_index[1].astype(jnp.int32),
    ])
    batch_i32 = user_batch.astype(jnp.int32)
    out, bvals = _gather_sc(user_x, idx_all, batch_i32)
    e0 = N_NODES
    sender_batch = bvals[e0:e0 + N_EDGES].astype(user_batch.dtype)
    receiver_batch = bvals[e0 + N_EDGES:e0 + 2 * N_EDGES].astype(user_batch.dtype)
    follower_batch = bvals[e0 + 2 * N_EDGES:].astype(user_batch.dtype)
    return out, sender_batch, receiver_batch, follower_batch
